# Initial kernel scaffold; baseline (speedup 1.0000x reference)
import jax
import jax.numpy as jnp
from jax.experimental import pallas as pl


def kernel(x, edge_index, batch, Wrel0, brel0, Wroot0, Wrel1, brel1, Wroot1, Wrel2, brel2, Wroot2):
    raise NotImplementedError("write your pallas kernel here")


# speedup vs baseline: 6.8848x; 6.8848x over previous
import functools

import jax
import jax.numpy as jnp
from jax import lax
from jax.experimental import pallas as pl
from jax.experimental.pallas import tpu as pltpu
from jax.experimental.pallas import tpu_sc as plsc

N = 10000
E = 320000
D_IN = 128
H = 64
D_OUT = 128

NC = 2
NS = 16
NW = NC * NS

B = 128
NK = 79
E_PAD = NW * NK * B
STRIPE = 640
NP = NS * STRIPE
DUMMY = N

_f32 = jnp.float32


def _sc_body(y_hbm, src_hbm, dst_hbm, part0, part1, src_v, dst_v, rows, acc,
             sem):
    c = lax.axis_index("c")
    s = lax.axis_index("s")
    wid = c * NS + s

    pltpu.sync_copy(src_hbm.at[wid], src_v)
    pltpu.sync_copy(dst_hbm.at[wid], dst_v)

    z = jnp.zeros((16,), _f32)

    def zrow(r, carry):
        for g in range(H // 16):
            rows[r, pl.ds(g * 16, 16)] = z
        return carry

    lax.fori_loop(0, B, zrow, 0)
    base = s * STRIPE
    for j in range(STRIPE // B):
        pltpu.sync_copy(rows, acc.at[pl.ds(base + j * B, B)])
    plsc.subcore_barrier()

    def chunk(k, carry):
        pltpu.async_copy(y_hbm.at[src_v.at[k]], rows, sem).wait()
        pltpu.sync_copy(rows, acc.at[dst_v.at[k]], add=True)
        return carry

    lax.fori_loop(0, NK, chunk, 0)
    plsc.subcore_barrier()

    @pl.when(c == 0)
    def _():
        pltpu.sync_copy(acc.at[pl.ds(base, STRIPE)], part0.at[pl.ds(base, STRIPE)])

    @pl.when(c == 1)
    def _():
        pltpu.sync_copy(acc.at[pl.ds(base, STRIPE)], part1.at[pl.ds(base, STRIPE)])


_sc_segsum = functools.partial(
    pl.kernel,
    out_type=(
        jax.ShapeDtypeStruct((NP, H), _f32),
        jax.ShapeDtypeStruct((NP, H), _f32),
    ),
    mesh=plsc.VectorSubcoreMesh(core_axis_name="c", subcore_axis_name="s"),
    compiler_params=pltpu.CompilerParams(use_tc_tiling_on_sc=False),
    scratch_types=[
        pltpu.VMEM((NK, B), jnp.int32),
        pltpu.VMEM((NK, B), jnp.int32),
        pltpu.VMEM((B, H), _f32),
        pltpu.VMEM_SHARED((NP, H), _f32),
        pltpu.SemaphoreType.DMA,
    ],
)(_sc_body)


_DN = (((1,), (1,)), ((), ()))


def _lrelu(v):
    return jnp.where(v >= 0, v, 0.01 * v)


def _t0_body(x_ref, wrel_ref, y_ref):
    y_ref[...] = lax.dot_general(x_ref[...], wrel_ref[...], _DN,
                                 preferred_element_type=_f32)


def _mid_body(x_ref, p0_ref, p1_ref, wroot_ref, b_ref, wrel_ref, h_ref, y_ref):
    agg = p0_ref[...] + p1_ref[...]
    root = lax.dot_general(x_ref[...], wroot_ref[...], _DN,
                           preferred_element_type=_f32)
    h = _lrelu(agg + b_ref[...] + root)
    h_ref[...] = h
    y_ref[...] = lax.dot_general(h, wrel_ref[...], _DN,
                                 preferred_element_type=_f32)


def _t2_body(x_ref, p0_ref, p1_ref, wroot_ref, b_ref, h_ref):
    agg = p0_ref[...] + p1_ref[...]
    root = lax.dot_general(x_ref[...], wroot_ref[...], _DN,
                           preferred_element_type=_f32)
    h_ref[...] = _lrelu(agg + b_ref[...] + root)


def _t3_body(x_ref, p0_ref, p1_ref, wrel_ref, b_ref, wroot_ref, out_ref):
    agg = p0_ref[...] + p1_ref[...]
    rel = lax.dot_general(agg, wrel_ref[...], _DN,
                          preferred_element_type=_f32)
    root = lax.dot_general(x_ref[...], wroot_ref[...], _DN,
                           preferred_element_type=_f32)
    out_ref[...] = _lrelu(rel + b_ref[...] + root)


_GRID = (NP // 1024,)


def _row_spec(w):
    return pl.BlockSpec((1024, w), lambda i: (i, 0))


def _full_spec(shape):
    return pl.BlockSpec(shape, lambda i: (0,) * len(shape))


def _tc_t0(x, wrel):
    return pl.pallas_call(
        _t0_body,
        grid=_GRID,
        in_specs=[_row_spec(D_IN), _full_spec(wrel.shape)],
        out_specs=_row_spec(H),
        out_shape=jax.ShapeDtypeStruct((NP, H), _f32),
    )(x, wrel)


def _tc_mid(x, p0, p1, wroot, b, wrel):
    return pl.pallas_call(
        _mid_body,
        grid=_GRID,
        in_specs=[_row_spec(x.shape[1]), _row_spec(H), _row_spec(H),
                  _full_spec(wroot.shape), _full_spec((1, H)),
                  _full_spec(wrel.shape)],
        out_specs=[_row_spec(H), _row_spec(H)],
        out_shape=[jax.ShapeDtypeStruct((NP, H), _f32),
                   jax.ShapeDtypeStruct((NP, H), _f32)],
    )(x, p0, p1, wroot, b.reshape(1, H), wrel)


def _tc_t2(x, p0, p1, wroot, b):
    return pl.pallas_call(
        _t2_body,
        grid=_GRID,
        in_specs=[_row_spec(H), _row_spec(H), _row_spec(H),
                  _full_spec(wroot.shape), _full_spec((1, H))],
        out_specs=_row_spec(H),
        out_shape=jax.ShapeDtypeStruct((NP, H), _f32),
    )(x, p0, p1, wroot, b.reshape(1, H))


def _tc_t3(x, p0, p1, wrel, b, wroot):
    return pl.pallas_call(
        _t3_body,
        grid=_GRID,
        in_specs=[_row_spec(H), _row_spec(H), _row_spec(H),
                  _full_spec(wrel.shape), _full_spec((1, D_OUT)),
                  _full_spec(wroot.shape)],
        out_specs=_row_spec(D_OUT),
        out_shape=jax.ShapeDtypeStruct((NP, D_OUT), _f32),
    )(x, p0, p1, wrel, b.reshape(1, D_OUT), wroot)


def kernel(x, edge_index, batch, Wrel0, brel0, Wroot0, Wrel1, brel1, Wroot1,
           Wrel2, brel2, Wroot2):
    pad = E_PAD - E
    src = jnp.concatenate([edge_index[0], jnp.zeros((pad,), jnp.int32)])
    dst = jnp.concatenate([edge_index[1], jnp.full((pad,), DUMMY, jnp.int32)])
    src3 = src.reshape(NW, NK, B)
    dst3 = dst.reshape(NW, NK, B)

    x_p = jnp.pad(x, ((0, NP - N), (0, 0)))

    y0 = _tc_t0(x_p, Wrel0)
    a0, b0 = _sc_segsum(y0, src3, dst3)
    h1, y1 = _tc_mid(x_p, a0, b0, Wroot0, brel0, Wrel1)
    a1, b1 = _sc_segsum(y1, src3, dst3)
    h2 = _tc_t2(h1, a1, b1, Wroot1, brel1)
    a2, b2 = _sc_segsum(h2, src3, dst3)
    out = _tc_t3(h2, a2, b2, Wrel2, brel2, Wroot2)
    return out[:N]
